# baseline (speedup 1.0000x reference)
import functools

import jax
import jax.numpy as jnp
from jax import lax
from jax.experimental import pallas as pl
from jax.experimental.pallas import tpu as pltpu
from jax.experimental.pallas import tpu_sc as plsc

_LANES = 16
_NUM_CORES = 2
_NUM_SUBCORES = 16


@functools.lru_cache(maxsize=None)
def _build_sc_kernel(N, Q, D, C, M):
    NW = _NUM_CORES * _NUM_SUBCORES
    TPW = N // NW
    S = TPW // C
    R = C * Q
    MT = M // _NUM_SUBCORES

    mesh = plsc.VectorSubcoreMesh(
        core_axis_name="c", subcore_axis_name="s",
        num_cores=_NUM_CORES, num_subcores=_NUM_SUBCORES)

    @functools.partial(
        pl.kernel,
        out_type=jax.ShapeDtypeStruct((N, D), jnp.float32),
        mesh=mesh,
        scratch_types=[
            pltpu.VMEM((TPW * Q,), jnp.int32),
            pltpu.VMEM((R, D // 2), jnp.int32),
            pltpu.VMEM((R, D // 2), jnp.int32),
            pltpu.VMEM((C, D), jnp.float32),
            pltpu.VMEM((C, D), jnp.float32),
            pltpu.VMEM_SHARED((M, D // 2), jnp.int32),
            pltpu.SemaphoreType.DMA,
            pltpu.SemaphoreType.DMA,
            pltpu.SemaphoreType.DMA,
            pltpu.SemaphoreType.DMA,
        ],
    )
    def sc_kernel(tab_hbm, fidx_hbm, out_hbm, fidx_v,
                  rows0, rows1, ob0, ob1, spm_tab, g0, g1, o0, o1):
        sid = lax.axis_index("s")
        wid = sid * _NUM_CORES + lax.axis_index("c")
        tbase = wid * TPW
        bufs = ((rows0, g0, ob0, o0), (rows1, g1, ob1, o1))

        pltpu.sync_copy(tab_hbm.at[pl.ds(sid * MT, MT)],
                        spm_tab.at[pl.ds(sid * MT, MT)])
        pltpu.sync_copy(fidx_hbm.at[pl.ds(tbase * Q, TPW * Q)], fidx_v)
        plsc.subcore_barrier()

        def gather(s, rows, gsem):
            return pltpu.async_copy(
                spm_tab.at[fidx_v.at[pl.ds(s * R, R)]], rows, gsem)

        himask = jnp.full((_LANES,), -65536, jnp.int32)

        def compute(rows_v, ob_v):
            def token(t, carry2):
                r0 = t * Q
                for jj in range(D // (2 * _LANES)):
                    sl = pl.ds(jj * _LANES, _LANES)
                    vs = [rows_v[r0 + q, sl] for q in range(Q)]
                    los = [lax.bitcast_convert_type(v << 16, jnp.float32)
                           for v in vs]
                    his = [lax.bitcast_convert_type(v & himask, jnp.float32)
                           for v in vs]
                    while len(los) > 1:
                        los = [los[i] + los[i + 1] for i in range(0, len(los), 2)]
                        his = [his[i] + his[i + 1] for i in range(0, len(his), 2)]
                    ob_v[t, pl.ds(jj * 2 * _LANES, _LANES)] = los[0]
                    ob_v[t, pl.ds(jj * 2 * _LANES + _LANES, _LANES)] = his[0]
                return carry2
            lax.fori_loop(0, C, token, 0)

        gather(0, rows0, g0)
        gather(1, rows1, g1)

        def body(i, carry):
            for b in range(2):
                rows, gsem, ob, osem = bufs[b]
                s = 2 * i + b
                pltpu.make_async_copy(
                    spm_tab.at[fidx_v.at[pl.ds(s * R, R)]], rows, gsem).wait()
                @pl.when(i > 0)
                def _():
                    pltpu.make_async_copy(
                        ob, out_hbm.at[pl.ds(tbase + (s - 2) * C, C)], osem).wait()
                compute(rows, ob)
                pltpu.async_copy(ob, out_hbm.at[pl.ds(tbase + s * C, C)], osem)
                @pl.when(s + 2 < S)
                def _():
                    gather(s + 2, rows, gsem)
            return carry

        lax.fori_loop(0, S // 2, body, 0)
        for b in range(2):
            s_last = S - 2 + b
            pltpu.make_async_copy(
                bufs[b][2], out_hbm.at[pl.ds(tbase + s_last * C, C)],
                bufs[b][3]).wait()

    return sc_kernel


def kernel(value, mask, tables):
    B, T, Q = value.shape
    Qt, K, D = tables.shape
    N = B * T

    v = value.reshape(N, Q).astype(jnp.int32)
    offs = (jnp.arange(Q, dtype=jnp.int32) * K)[None, :]
    fidx = jnp.where(mask.reshape(N, 1), v + offs, Q * K).reshape(N * Q)
    tab = tables.reshape(Q * K, D).astype(jnp.bfloat16)
    tab = tab.reshape(Q * K, D // 32, 2, 16).transpose(0, 1, 3, 2)
    tab = lax.bitcast_convert_type(tab, jnp.int32).reshape(Q * K, D // 2)
    M = (Q * K + 8 + 127) // 128 * 128
    tab = jnp.concatenate(
        [tab, jnp.zeros((M - Q * K, D // 2), jnp.int32)], axis=0)

    out = _build_sc_kernel(N, Q, D, 16, M)(tab, fidx)
    return out.reshape(B, T, D)


# speedup vs baseline: 1.0163x; 1.0163x over previous
import functools

import jax
import jax.numpy as jnp
from jax import lax
from jax.experimental import pallas as pl
from jax.experimental.pallas import tpu as pltpu
from jax.experimental.pallas import tpu_sc as plsc

_LANES = 16
_NUM_CORES = 2
_NUM_SUBCORES = 16
_MAX_IDX = 128


@functools.lru_cache(maxsize=None)
def _build_sc_kernel(N, Q, D, C):
    NW = _NUM_CORES * _NUM_SUBCORES
    TPW = N // NW
    S = TPW // C
    R = C * Q
    G = R // _MAX_IDX

    mesh = plsc.VectorSubcoreMesh(
        core_axis_name="c", subcore_axis_name="s",
        num_cores=_NUM_CORES, num_subcores=_NUM_SUBCORES)

    @functools.partial(
        pl.kernel,
        out_type=jax.ShapeDtypeStruct((N, D), jnp.float32),
        mesh=mesh,
        scratch_types=[
            pltpu.VMEM((TPW * Q,), jnp.int32),
            pltpu.VMEM((R, D // 2), jnp.int32),
            pltpu.VMEM((R, D // 2), jnp.int32),
            pltpu.VMEM((C, D), jnp.float32),
            pltpu.VMEM((C, D), jnp.float32),
            pltpu.SemaphoreType.DMA,
            pltpu.SemaphoreType.DMA,
            pltpu.SemaphoreType.DMA,
            pltpu.SemaphoreType.DMA,
        ],
    )
    def sc_kernel(tab_hbm, fidx_hbm, out_hbm, fidx_v,
                  rows0, rows1, ob0, ob1, g0, g1, o0, o1):
        wid = lax.axis_index("s") * _NUM_CORES + lax.axis_index("c")
        tbase = wid * TPW
        bufs = ((rows0, g0, ob0, o0), (rows1, g1, ob1, o1))

        pltpu.sync_copy(fidx_hbm.at[pl.ds(tbase * Q, TPW * Q)], fidx_v)

        def gather(s, rows, gsem):
            for h in range(G):
                pltpu.async_copy(
                    tab_hbm.at[fidx_v.at[pl.ds(s * R + h * _MAX_IDX, _MAX_IDX)]],
                    rows.at[pl.ds(h * _MAX_IDX, _MAX_IDX)], gsem)

        def gather_wait(s, rows, gsem):
            for h in range(G):
                pltpu.make_async_copy(
                    tab_hbm.at[fidx_v.at[pl.ds(s * R + h * _MAX_IDX, _MAX_IDX)]],
                    rows.at[pl.ds(h * _MAX_IDX, _MAX_IDX)], gsem).wait()

        himask = jnp.full((_LANES,), -65536, jnp.int32)

        def compute(rows_v, ob_v):
            def token(t, carry2):
                r0 = t * Q
                for jj in range(D // (2 * _LANES)):
                    sl = pl.ds(jj * _LANES, _LANES)
                    vs = [rows_v[r0 + q, sl] for q in range(Q)]
                    los = [lax.bitcast_convert_type(v << 16, jnp.float32)
                           for v in vs]
                    his = [lax.bitcast_convert_type(v & himask, jnp.float32)
                           for v in vs]
                    while len(los) > 1:
                        los = [los[i] + los[i + 1] for i in range(0, len(los), 2)]
                        his = [his[i] + his[i + 1] for i in range(0, len(his), 2)]
                    ob_v[t, pl.ds(jj * 2 * _LANES, _LANES)] = los[0]
                    ob_v[t, pl.ds(jj * 2 * _LANES + _LANES, _LANES)] = his[0]
                return carry2
            lax.fori_loop(0, C, token, 0)

        gather(0, rows0, g0)
        gather(1, rows1, g1)

        def body(i, carry):
            for b in range(2):
                rows, gsem, ob, osem = bufs[b]
                s = 2 * i + b
                gather_wait(s, rows, gsem)
                @pl.when(i > 0)
                def _():
                    pltpu.make_async_copy(
                        ob, out_hbm.at[pl.ds(tbase + (s - 2) * C, C)], osem).wait()
                compute(rows, ob)
                pltpu.async_copy(ob, out_hbm.at[pl.ds(tbase + s * C, C)], osem)
                @pl.when(s + 2 < S)
                def _():
                    gather(s + 2, rows, gsem)
            return carry

        lax.fori_loop(0, S // 2, body, 0)
        for b in range(2):
            s_last = S - 2 + b
            pltpu.make_async_copy(
                bufs[b][2], out_hbm.at[pl.ds(tbase + s_last * C, C)],
                bufs[b][3]).wait()

    return sc_kernel


def kernel(value, mask, tables):
    B, T, Q = value.shape
    Qt, K, D = tables.shape
    N = B * T

    v = value.reshape(N, Q).astype(jnp.int32)
    offs = (jnp.arange(Q, dtype=jnp.int32) * K)[None, :]
    fidx = jnp.where(mask.reshape(N, 1), v + offs, Q * K).reshape(N * Q)
    tab = tables.reshape(Q * K, D).astype(jnp.bfloat16)
    tab = tab.reshape(Q * K, D // 32, 2, 16).transpose(0, 1, 3, 2)
    tab = lax.bitcast_convert_type(tab, jnp.int32).reshape(Q * K, D // 2)
    tab = jnp.concatenate([tab, jnp.zeros((8, D // 2), jnp.int32)], axis=0)

    out = _build_sc_kernel(N, Q, D, 32)(tab, fidx)
    return out.reshape(B, T, D)


# speedup vs baseline: 1.2774x; 1.2570x over previous
import functools

import jax
import jax.numpy as jnp
from jax import lax
from jax.experimental import pallas as pl
from jax.experimental.pallas import tpu as pltpu
from jax.experimental.pallas import tpu_sc as plsc

_LANES = 16
_NUM_CORES = 2
_NUM_SUBCORES = 16
_MAX_IDX = 128


@functools.lru_cache(maxsize=None)
def _build_sc_kernel(N, Q, D, C):
    NW = _NUM_CORES * _NUM_SUBCORES
    TPW = N // NW
    S = TPW // C
    R = C * Q
    G = R // _MAX_IDX

    mesh = plsc.VectorSubcoreMesh(
        core_axis_name="c", subcore_axis_name="s",
        num_cores=_NUM_CORES, num_subcores=_NUM_SUBCORES)

    @functools.partial(
        pl.kernel,
        out_type=jax.ShapeDtypeStruct((N, D), jnp.float32),
        mesh=mesh,
        scratch_types=[
            pltpu.VMEM((TPW * Q,), jnp.int32),
            pltpu.VMEM((R, D // 2), jnp.int32),
            pltpu.VMEM((R, D // 2), jnp.int32),
            pltpu.VMEM((C, D), jnp.float32),
            pltpu.VMEM((C, D), jnp.float32),
            pltpu.SemaphoreType.DMA,
            pltpu.SemaphoreType.DMA,
            pltpu.SemaphoreType.DMA,
            pltpu.SemaphoreType.DMA,
        ],
    )
    def sc_kernel(tab_hbm, fidx_hbm, out_hbm, fidx_v,
                  rows0, rows1, ob0, ob1, g0, g1, o0, o1):
        wid = lax.axis_index("s") * _NUM_CORES + lax.axis_index("c")
        tbase = wid * TPW
        bufs = ((rows0, g0, ob0, o0), (rows1, g1, ob1, o1))

        pltpu.sync_copy(fidx_hbm.at[pl.ds(tbase * Q, TPW * Q)], fidx_v)

        def gather(s, rows, gsem):
            for h in range(G):
                pltpu.async_copy(
                    tab_hbm.at[fidx_v.at[pl.ds(s * R + h * _MAX_IDX, _MAX_IDX)]],
                    rows.at[pl.ds(h * _MAX_IDX, _MAX_IDX)], gsem)

        def gather_wait(s, rows, gsem):
            for h in range(G):
                pltpu.make_async_copy(
                    tab_hbm.at[fidx_v.at[pl.ds(s * R + h * _MAX_IDX, _MAX_IDX)]],
                    rows.at[pl.ds(h * _MAX_IDX, _MAX_IDX)], gsem).wait()

        himask = jnp.full((_LANES,), -65536, jnp.int32)

        def compute(rows_v, ob_v):
            @plsc.parallel_loop(0, C, unroll=2)
            def token(t):
                r0 = t * Q
                for jj in range(D // (2 * _LANES)):
                    sl = pl.ds(jj * _LANES, _LANES)
                    vs = [rows_v[r0 + q, sl] for q in range(Q)]
                    los = [lax.bitcast_convert_type(v << 16, jnp.float32)
                           for v in vs]
                    his = [lax.bitcast_convert_type(v & himask, jnp.float32)
                           for v in vs]
                    while len(los) > 1:
                        los = [los[i] + los[i + 1] for i in range(0, len(los), 2)]
                        his = [his[i] + his[i + 1] for i in range(0, len(his), 2)]
                    ob_v[t, pl.ds(jj * 2 * _LANES, _LANES)] = los[0]
                    ob_v[t, pl.ds(jj * 2 * _LANES + _LANES, _LANES)] = his[0]

        gather(0, rows0, g0)
        gather(1, rows1, g1)

        def body(i, carry):
            for b in range(2):
                rows, gsem, ob, osem = bufs[b]
                s = 2 * i + b
                gather_wait(s, rows, gsem)
                @pl.when(i > 0)
                def _():
                    pltpu.make_async_copy(
                        ob, out_hbm.at[pl.ds(tbase + (s - 2) * C, C)], osem).wait()
                compute(rows, ob)
                pltpu.async_copy(ob, out_hbm.at[pl.ds(tbase + s * C, C)], osem)
                @pl.when(s + 2 < S)
                def _():
                    gather(s + 2, rows, gsem)
            return carry

        lax.fori_loop(0, S // 2, body, 0)
        for b in range(2):
            s_last = S - 2 + b
            pltpu.make_async_copy(
                bufs[b][2], out_hbm.at[pl.ds(tbase + s_last * C, C)],
                bufs[b][3]).wait()

    return sc_kernel


def kernel(value, mask, tables):
    B, T, Q = value.shape
    Qt, K, D = tables.shape
    N = B * T

    v = value.reshape(N, Q).astype(jnp.int32)
    offs = (jnp.arange(Q, dtype=jnp.int32) * K)[None, :]
    fidx = jnp.where(mask.reshape(N, 1), v + offs, Q * K).reshape(N * Q)
    tab = tables.reshape(Q * K, D).astype(jnp.bfloat16)
    tab = tab.reshape(Q * K, D // 32, 2, 16).transpose(0, 1, 3, 2)
    tab = lax.bitcast_convert_type(tab, jnp.int32).reshape(Q * K, D // 2)
    tab = jnp.concatenate([tab, jnp.zeros((8, D // 2), jnp.int32)], axis=0)

    out = _build_sc_kernel(N, Q, D, 32)(tab, fidx)
    return out.reshape(B, T, D)
